# baseline (speedup 1.0000x reference)
import functools

import jax
import jax.numpy as jnp
from jax import lax
from jax.experimental import pallas as pl
from jax.experimental.pallas import tpu as pltpu
from jax.experimental.pallas import tpu_sc as plsc

VOCAB = 1000000
L_CTX = 200
D = 32
BATCH = 4096

NC = 2
NS = 16
NW = NC * NS

N = BATCH * L_CTX
R_PER_W = N // NW
C = 1600
N_CHUNKS = R_PER_W // C

NB_TILES = BATCH // 128
N_BLOCKS = L_CTX * NB_TILES
BLK_PER_W = N_BLOCKS // NW


def _make_gather():
    mesh = plsc.VectorSubcoreMesh(
        core_axis_name="c", subcore_axis_name="s",
        num_cores=NC, num_subcores=NS)

    @functools.partial(
        pl.kernel,
        out_type=jax.ShapeDtypeStruct((N, D), jnp.float32),
        mesh=mesh,
        scratch_types=[
            pltpu.VMEM((C,), jnp.int32),
            pltpu.VMEM((C, D), jnp.float32),
            pltpu.SemaphoreType.DMA,
        ],
        compiler_params=pltpu.CompilerParams(use_tc_tiling_on_sc=False),
    )
    def gather_kernel(x_hbm, tok_hbm, out_hbm, idx_v, rows_v, sem):
        wid = lax.axis_index("s") * NC + lax.axis_index("c")
        base = wid * R_PER_W

        def chunk_body(ci, _):
            cb = base + ci * C
            pltpu.sync_copy(x_hbm.at[pl.ds(cb, C)], idx_v)
            pltpu.async_copy(tok_hbm.at[idx_v], rows_v, sem).wait()
            pltpu.sync_copy(rows_v, out_hbm.at[pl.ds(cb, C)])
            return 0

        lax.fori_loop(0, N_CHUNKS, chunk_body, 0)

    return gather_kernel


def _make_transpose():
    mesh = plsc.VectorSubcoreMesh(
        core_axis_name="c", subcore_axis_name="s",
        num_cores=NC, num_subcores=NS)

    @functools.partial(
        pl.kernel,
        out_type=jax.ShapeDtypeStruct((L_CTX * D, BATCH), jnp.float32),
        mesh=mesh,
        scratch_types=[
            pltpu.VMEM((128 * D,), jnp.float32),
            pltpu.VMEM((D, 128), jnp.float32),
            pltpu.VMEM((L_CTX * D,), jnp.float32),
        ],
        compiler_params=pltpu.CompilerParams(
            use_tc_tiling_on_sc=True, needs_layout_passes=False),
    )
    def transpose_kernel(rows_hbm, pos_hbm, out_hbm, bin_v, bout_v, pos_v):
        wid = lax.axis_index("s") * NC + lax.axis_index("c")
        pltpu.sync_copy(pos_hbm, pos_v)
        lane = lax.iota(jnp.int32, 16)
        idx_base = lane * D

        def blk_body(t, _):
            j = wid * BLK_PER_W + t
            l = j // NB_TILES
            bt = j - l * NB_TILES
            src = pl.multiple_of((l * BATCH + bt * 128) * D, 128 * D)
            pltpu.sync_copy(rows_hbm.at[pl.ds(src, 128 * D)], bin_v)

            def d_body(d, _):
                pidx = jnp.broadcast_to(l * D + d, (16,))
                pval = plsc.load_gather(pos_v, [pidx])

                def g_body(g, _):
                    idx = idx_base + (g * 16 * D + d)
                    vals = plsc.load_gather(bin_v, [idx])
                    bout_v[d, pl.ds(g * 16, 16)] = vals + pval
                    return 0

                lax.fori_loop(0, 8, g_body, 0, unroll=True)
                return 0

            lax.fori_loop(0, D, d_body, 0)
            orow = pl.multiple_of(l * D, D)
            ocol = pl.multiple_of(bt * 128, 128)
            pltpu.sync_copy(bout_v,
                            out_hbm.at[pl.ds(orow, D), pl.ds(ocol, 128)])
            return 0

        lax.fori_loop(0, BLK_PER_W, blk_body, 0)

    return transpose_kernel


_gather_kernel = _make_gather()
_transpose_kernel = _make_transpose()


@jax.jit
def kernel(x, token_table, pos_table):
    x_t = x.T.reshape(N).astype(jnp.int32)
    tok_flat = lax.optimization_barrier(token_table.reshape(VOCAB * D))
    tok_lin = tok_flat.reshape(VOCAB, D)
    rows = _gather_kernel(x_t, tok_lin)
    rows_flat = rows.reshape(N * D)
    pos_flat = pos_table.reshape(L_CTX * D)
    out2d = _transpose_kernel(rows_flat, pos_flat)
    return out2d.reshape(L_CTX, D, BATCH).transpose(2, 0, 1)


# speedup vs baseline: 1.4530x; 1.4530x over previous
import functools

import jax
import jax.numpy as jnp
from jax import lax
from jax.experimental import pallas as pl
from jax.experimental.pallas import tpu as pltpu
from jax.experimental.pallas import tpu_sc as plsc

VOCAB = 1000000
L_CTX = 200
D = 32
BATCH = 4096

NC = 2
NS = 16
NW = NC * NS

N = BATCH * L_CTX
R_PER_W = N // NW
C = 1600
N_CHUNKS = R_PER_W // C


def _make_gather():
    mesh = plsc.VectorSubcoreMesh(
        core_axis_name="c", subcore_axis_name="s",
        num_cores=NC, num_subcores=NS)

    @functools.partial(
        pl.kernel,
        out_type=jax.ShapeDtypeStruct((N, D), jnp.float32),
        mesh=mesh,
        scratch_types=[
            pltpu.VMEM((C,), jnp.int32),
            pltpu.VMEM((C, D), jnp.float32),
            pltpu.SemaphoreType.DMA,
        ],
        compiler_params=pltpu.CompilerParams(use_tc_tiling_on_sc=False),
    )
    def gather_kernel(x_hbm, tok_hbm, out_hbm, idx_v, rows_v, sem):
        wid = lax.axis_index("s") * NC + lax.axis_index("c")
        base = wid * R_PER_W

        def chunk_body(ci, _):
            cb = base + ci * C
            pltpu.sync_copy(x_hbm.at[pl.ds(cb, C)], idx_v)
            pltpu.async_copy(tok_hbm.at[idx_v], rows_v, sem).wait()
            pltpu.sync_copy(rows_v, out_hbm.at[pl.ds(cb, C)])
            return 0

        lax.fori_loop(0, N_CHUNKS, chunk_body, 0)

    return gather_kernel


_gather_kernel = _make_gather()


@jax.jit
def kernel(x, token_table, pos_table):
    x_t = x.T.reshape(N).astype(jnp.int32)
    tok128 = lax.optimization_barrier(
        token_table.reshape(VOCAB * D // 128, 128))
    tok_lin = tok128.reshape(VOCAB, D)
    rows = _gather_kernel(x_t, tok_lin)
    out_t = rows.reshape(L_CTX, BATCH, D)
    return out_t.transpose(1, 0, 2) + pos_table[None, :, :]
